# baseline (speedup 1.0000x reference)
import jax
import jax.numpy as jnp
from jax.experimental import pallas as pl
from jax.experimental.pallas import tpu as pltpu

_DMAX = 3
_NSLOTS = 9


def _p1_col(j, s, nb):
    shifted = s + (s >= j).astype(s.dtype)
    return jnp.where(s == nb - 1, j, shifted)


def _slot(d, row):
    return jnp.where(d == 1, row % 2,
                     jnp.where(d == 2, 2 + row % 3, 5 + row % 4))


def _merged_kernel(L_ref, x0bf_ref, WAb_ref, WBb_ref, WC_ref, bias_ref,
                   o_ref, xcat, xacc, pacc, ust):
    nb = pl.num_programs(0)
    j = pl.program_id(0)
    s = pl.program_id(1)
    bm = L_ref.shape[0]
    C = x0bf_ref.shape[1]

    def _combine(r):
        x0b = xcat[pl.ds(r * bm, bm), :C]
        x1b = xcat[pl.ds(r * bm, bm), C:]
        o_ref[...] = (
            jnp.dot(x0b, WAb_ref[...], preferred_element_type=jnp.float32)
            + jnp.dot(x1b, WBb_ref[...], preferred_element_type=jnp.float32)
            + jnp.dot(pacc[pl.ds(r * bm, bm), :], WC_ref[...],
                      preferred_element_type=jnp.float32)
            + bias_ref[...]
        )

    @pl.when(jnp.logical_and(j == 0, s == 0))
    def _zero_pacc():
        pacc[...] = jnp.zeros_like(pacc)

    @pl.when(s < nb)
    def _pass1_step():
        c = _p1_col(j, s, nb)

        @pl.when(j == 0)
        def _seed_x0():
            xcat[pl.ds(c * bm, bm), :C] = x0bf_ref[pl.ds(c * bm, bm), :]

        Lb = L_ref[...].astype(jnp.bfloat16)
        rhs = xcat[pl.ds(c * bm, bm), :]
        prod = jnp.dot(Lb, rhs, preferred_element_type=jnp.float32)

        @pl.when(s == 0)
        def _x1_init():
            xacc[...] = prod[:, :C]

        @pl.when(s > 0)
        def _x1_accum():
            xacc[...] += prod[:, :C]

        @pl.when(c < j)
        def _lower():
            pacc[pl.ds(j * bm, bm), :] += prod[:, C:]

        d_up = c - j

        @pl.when(jnp.logical_and(d_up >= 1, d_up <= _DMAX))
        def _stash_near():
            ust[pl.ds(_slot(d_up, j) * bm, bm), :] = Lb

        @pl.when(s == nb - 1)
        def _diag_finish():
            x1b16 = xacc[...].astype(jnp.bfloat16)
            xcat[pl.ds(j * bm, bm), C:] = x1b16
            pacc[pl.ds(j * bm, bm), :] += jnp.dot(
                Lb, x1b16, preferred_element_type=jnp.float32)

            @pl.when(j == nb - 1)
            def _finalize_last_strip():
                _combine(nb - 1)

    @pl.when(s >= nb)
    def _upper_step():
        i = s - nb
        d = j - i

        @pl.when(jnp.logical_and(i < j, d <= _DMAX))
        def _accum_from_stash():
            x1j = xcat[pl.ds(j * bm, bm), C:]
            lhs = ust[pl.ds(_slot(d, i) * bm, bm), :]
            pacc[pl.ds(i * bm, bm), :] += jnp.dot(
                lhs, x1j, preferred_element_type=jnp.float32)

        @pl.when(jnp.logical_and(i < j, d > _DMAX))
        def _accum_from_hbm():
            x1j = xcat[pl.ds(j * bm, bm), C:]
            pacc[pl.ds(i * bm, bm), :] += jnp.dot(
                L_ref[...].astype(jnp.bfloat16), x1j,
                preferred_element_type=jnp.float32)

        @pl.when(jnp.logical_and(i < j, j == nb - 1))
        def _finalize_strip_i():
            _combine(i)


def kernel(x, L, weight, bias):
    N, M, Fin = x.shape
    Fout = weight.shape[1]
    x0 = jnp.transpose(x, (1, 0, 2)).reshape(M, N * Fin)
    x0bf = x0.astype(jnp.bfloat16)

    W = weight.reshape(Fin, 3, Fout)
    eyeN = jnp.eye(N, dtype=weight.dtype)
    W0e = jnp.kron(eyeN, W[:, 0, :])
    W1e = jnp.kron(eyeN, W[:, 1, :])
    W2e = jnp.kron(eyeN, W[:, 2, :])
    WAb = (W0e - W2e).astype(jnp.bfloat16)
    WBb = W1e.astype(jnp.bfloat16)
    WC = 2.0 * W2e
    bias_row = jnp.tile(bias, N).reshape(1, N * Fout)

    bm = 1024
    nb = M // bm
    C = N * Fin
    Co = N * Fout

    def _L_index(j, s):
        p1 = (j, _p1_col(j, s, nb))
        i_eff = jnp.minimum(s - nb, j - 1)
        i2 = jnp.minimum(i_eff, j - _DMAX - 1)
        up_row = jnp.where(i2 < 0, j, i2)
        return (jnp.where(s < nb, p1[0], up_row),
                jnp.where(s < nb, p1[1], j))

    def _o_index(j, s):
        return (jnp.where(jnp.logical_or(j < nb - 1, s < nb),
                          nb - 1, s - nb), 0)

    out_flat = pl.pallas_call(
        _merged_kernel,
        grid=(nb, 2 * nb - 1),
        in_specs=[
            pl.BlockSpec((bm, bm), _L_index),
            pl.BlockSpec((M, C), lambda j, s: (0, 0)),
            pl.BlockSpec((C, Co), lambda j, s: (0, 0)),
            pl.BlockSpec((C, Co), lambda j, s: (0, 0)),
            pl.BlockSpec((C, Co), lambda j, s: (0, 0)),
            pl.BlockSpec((1, Co), lambda j, s: (0, 0)),
        ],
        out_specs=pl.BlockSpec((bm, Co), _o_index),
        out_shape=jax.ShapeDtypeStruct((M, Co), jnp.float32),
        scratch_shapes=[
            pltpu.VMEM((M, 2 * C), jnp.bfloat16),
            pltpu.VMEM((bm, C), jnp.float32),
            pltpu.VMEM((M, C), jnp.float32),
            pltpu.VMEM((_NSLOTS * bm, bm), jnp.bfloat16),
        ],
    )(L, x0bf, WAb, WBb, WC, bias_row)

    return out_flat.reshape(M, N, Fout).transpose(1, 0, 2)


# speedup vs baseline: 1.1689x; 1.1689x over previous
import jax
import jax.numpy as jnp
from jax.experimental import pallas as pl
from jax.experimental.pallas import tpu as pltpu


def _p1_col(j, s, nb):
    shifted = s + (s >= j).astype(s.dtype)
    return jnp.where(s == nb - 1, j, shifted)


def _merged_kernel(L_ref, x0bf_ref, WAb_ref, WBb_ref, WC_ref, bias_ref,
                   o_ref, xcat, xacc, pacc):
    nb = pl.num_programs(0)
    j = pl.program_id(0)
    s = pl.program_id(1)
    bm = L_ref.shape[0]
    C = x0bf_ref.shape[1]

    def _combine(r):
        x0b = xcat[pl.ds(r * bm, bm), :C]
        x1b = xcat[pl.ds(r * bm, bm), C:]
        o_ref[...] = (
            jnp.dot(x0b, WAb_ref[...], preferred_element_type=jnp.float32)
            + jnp.dot(x1b, WBb_ref[...], preferred_element_type=jnp.float32)
            + jnp.dot(pacc[pl.ds(r * bm, bm), :], WC_ref[...],
                      preferred_element_type=jnp.float32)
            + bias_ref[...]
        )

    @pl.when(jnp.logical_and(j == 0, s == 0))
    def _zero_pacc():
        pacc[...] = jnp.zeros_like(pacc)

    @pl.when(s < nb)
    def _pass1_step():
        c = _p1_col(j, s, nb)

        @pl.when(j == 0)
        def _seed_x0():
            xcat[pl.ds(c * bm, bm), :C] = x0bf_ref[pl.ds(c * bm, bm), :]

        Lb = L_ref[...].astype(jnp.bfloat16)
        rhs = xcat[pl.ds(c * bm, bm), :]
        prod = jnp.dot(Lb, rhs, preferred_element_type=jnp.float32)

        @pl.when(s == 0)
        def _x1_init():
            xacc[...] = prod[:, :C]

        @pl.when(s > 0)
        def _x1_accum():
            xacc[...] += prod[:, :C]

        @pl.when(c < j)
        def _lower():
            pacc[pl.ds(j * bm, bm), :] += prod[:, C:]

        @pl.when(s == nb - 1)
        def _diag_finish():
            x1b16 = xacc[...].astype(jnp.bfloat16)
            xcat[pl.ds(j * bm, bm), C:] = x1b16
            pacc[pl.ds(j * bm, bm), :] += jnp.dot(
                Lb, x1b16, preferred_element_type=jnp.float32)

            @pl.when(j == nb - 1)
            def _finalize_last_strip():
                _combine(nb - 1)

    @pl.when(s >= nb)
    def _upper_step():
        i = s - nb

        @pl.when(i < j)
        def _accum_upper():
            x1j = xcat[pl.ds(j * bm, bm), C:]
            pacc[pl.ds(i * bm, bm), :] += jnp.dot(
                L_ref[...].astype(jnp.bfloat16), x1j,
                preferred_element_type=jnp.float32)

            @pl.when(j == nb - 1)
            def _finalize_strip_i():
                _combine(i)


def kernel(x, L, weight, bias):
    N, M, Fin = x.shape
    Fout = weight.shape[1]
    x0 = jnp.transpose(x, (1, 0, 2)).reshape(M, N * Fin)
    x0bf = x0.astype(jnp.bfloat16)

    W = weight.reshape(Fin, 3, Fout)
    eyeN = jnp.eye(N, dtype=weight.dtype)
    W0e = jnp.kron(eyeN, W[:, 0, :])
    W1e = jnp.kron(eyeN, W[:, 1, :])
    W2e = jnp.kron(eyeN, W[:, 2, :])
    WAb = (W0e - W2e).astype(jnp.bfloat16)
    WBb = W1e.astype(jnp.bfloat16)
    WC = 2.0 * W2e
    bias_row = jnp.tile(bias, N).reshape(1, N * Fout)

    bm = 2048
    nb = M // bm
    C = N * Fin
    Co = N * Fout

    def _L_index(j, s):
        i = s - nb
        p1 = (j, _p1_col(j, s, nb))
        up_row = jnp.where(i < j, i, jnp.maximum(j - 1, 0))
        up_col = jnp.where(j > 0, j, 0)
        return (jnp.where(s < nb, p1[0], up_row),
                jnp.where(s < nb, p1[1], up_col))

    def _o_index(j, s):
        return (jnp.where(jnp.logical_or(j < nb - 1, s < nb),
                          nb - 1, s - nb), 0)

    out_flat = pl.pallas_call(
        _merged_kernel,
        grid=(nb, 2 * nb - 1),
        in_specs=[
            pl.BlockSpec((bm, bm), _L_index),
            pl.BlockSpec((M, C), lambda j, s: (0, 0)),
            pl.BlockSpec((C, Co), lambda j, s: (0, 0)),
            pl.BlockSpec((C, Co), lambda j, s: (0, 0)),
            pl.BlockSpec((C, Co), lambda j, s: (0, 0)),
            pl.BlockSpec((1, Co), lambda j, s: (0, 0)),
        ],
        out_specs=pl.BlockSpec((bm, Co), _o_index),
        out_shape=jax.ShapeDtypeStruct((M, Co), jnp.float32),
        scratch_shapes=[
            pltpu.VMEM((M, 2 * C), jnp.bfloat16),
            pltpu.VMEM((bm, C), jnp.float32),
            pltpu.VMEM((M, C), jnp.float32),
        ],
    )(L, x0bf, WAb, WBb, WC, bias_row)

    return out_flat.reshape(M, N, Fout).transpose(1, 0, 2)
